# Initial kernel scaffold; baseline (speedup 1.0000x reference)
import jax
import jax.numpy as jnp
from jax.experimental import pallas as pl


def kernel(hidden_states, W_router, b_router, W1, b1, W2, b2):
    raise NotImplementedError("write your pallas kernel here")


# speedup vs baseline: 1.0186x; 1.0186x over previous
import functools

import jax
import jax.numpy as jnp
from jax.experimental import pallas as pl
from jax.experimental.pallas import tpu as pltpu

_NE = 8
_LANES = 128


def _router_body(x_ref, wr_ref, br_ref,
                 logits_ref, rw_ref, dm_ref, eidx_ref, rank_ref, gate_ref,
                 counts_ref, aux_ref,
                 cnt_s, arw_s, adm_s, *, n_tokens):
    i = pl.program_id(0)
    nb = pl.num_programs(0)

    @pl.when(i == 0)
    def _():
        cnt_s[...] = jnp.zeros_like(cnt_s)
        arw_s[...] = jnp.zeros_like(arw_s)
        adm_s[...] = jnp.zeros_like(adm_s)

    x = x_ref[...]
    logits = jnp.dot(x, wr_ref[...], preferred_element_type=jnp.float32)
    logits = logits + br_ref[...]
    logits_ref[...] = logits

    lane = jax.lax.broadcasted_iota(jnp.int32, logits.shape, 1)
    valid = lane < _NE
    neg = jnp.float32(-1e30)
    lm = jnp.where(valid, logits, neg)

    mx = jnp.max(lm, axis=1, keepdims=True)
    ex = jnp.where(valid, jnp.exp(lm - mx), 0.0)
    rw = ex / jnp.sum(ex, axis=1, keepdims=True)
    rw_ref[...] = rw

    big = jnp.int32(1000)
    v1 = mx
    i1 = jnp.min(jnp.where(lm == v1, lane, big), axis=1, keepdims=True)
    lm2 = jnp.where(lane == i1, neg, lm)
    v2 = jnp.max(lm2, axis=1, keepdims=True)
    i2 = jnp.min(jnp.where(lm2 == v2, lane, big), axis=1, keepdims=True)

    e2 = jnp.exp(v2 - v1)
    g1 = 1.0 / (1.0 + e2)
    g2 = e2 / (1.0 + e2)

    dm = jnp.where(lane == i1, g1, 0.0) + jnp.where(lane == i2, g2, 0.0)
    dm_ref[...] = dm

    m01 = (jnp.where(lane == i1, 1, 0) + jnp.where(lane == i2, 1, 0)).astype(jnp.int32)
    r_io = jax.lax.broadcasted_iota(jnp.int32, (m01.shape[0], m01.shape[0]), 0)
    c_io = jax.lax.broadcasted_iota(jnp.int32, (m01.shape[0], m01.shape[0]), 1)
    tril = (c_io <= r_io).astype(jnp.float32)
    incl = jnp.dot(tril, m01.astype(jnp.float32),
                   preferred_element_type=jnp.float32).astype(jnp.int32)
    rankm = cnt_s[...] + incl - 1
    r1 = jnp.sum(jnp.where(lane == i1, rankm, 0), axis=1, keepdims=True)
    r2 = jnp.sum(jnp.where(lane == i2, rankm, 0), axis=1, keepdims=True)
    cnt_s[...] = cnt_s[...] + jnp.sum(m01, axis=0, keepdims=True)

    zi = jnp.zeros_like(lane)
    eidx_ref[...] = jnp.where(lane == 0, i1, jnp.where(lane == 1, i2, zi))
    rank_ref[...] = jnp.where(lane == 0, r1, jnp.where(lane == 1, r2, zi))
    gate_ref[...] = jnp.where(lane == 0, g1, jnp.where(lane == 1, g2, 0.0))

    arw_s[...] = arw_s[...] + jnp.sum(rw, axis=0, keepdims=True)
    adm_s[...] = adm_s[...] + jnp.sum(dm, axis=0, keepdims=True)

    @pl.when(i == nb - 1)
    def _():
        counts_ref[...] = cnt_s[...]
        scale = jnp.float32(_NE) / jnp.float32(n_tokens * n_tokens)
        aux_ref[...] = jnp.sum(arw_s[...] * adm_s[...],
                               axis=1, keepdims=True) * scale


def _run_router(flat, wr_pad, br_pad):
    n, d = flat.shape
    bm = min(512, n)
    nb = n // bm
    f32 = jnp.float32
    i32 = jnp.int32
    out_shapes = (
        jax.ShapeDtypeStruct((n, _LANES), f32),
        jax.ShapeDtypeStruct((n, _LANES), f32),
        jax.ShapeDtypeStruct((n, _LANES), f32),
        jax.ShapeDtypeStruct((n, _LANES), i32),
        jax.ShapeDtypeStruct((n, _LANES), i32),
        jax.ShapeDtypeStruct((n, _LANES), f32),
        jax.ShapeDtypeStruct((1, _LANES), i32),
        jax.ShapeDtypeStruct((1, 1), f32),
    )
    tok_spec = pl.BlockSpec((bm, _LANES), lambda i: (i, 0))
    out_specs = [tok_spec] * 6 + [
        pl.BlockSpec((1, _LANES), lambda i: (0, 0)),
        pl.BlockSpec((1, 1), lambda i: (0, 0)),
    ]
    return pl.pallas_call(
        functools.partial(_router_body, n_tokens=n),
        grid=(nb,),
        in_specs=[
            pl.BlockSpec((bm, d), lambda i: (i, 0)),
            pl.BlockSpec((d, _LANES), lambda i: (0, 0)),
            pl.BlockSpec((1, _LANES), lambda i: (0, 0)),
        ],
        out_specs=out_specs,
        out_shape=out_shapes,
        scratch_shapes=[
            pltpu.VMEM((1, _LANES), i32),
            pltpu.VMEM((1, _LANES), f32),
            pltpu.VMEM((1, _LANES), f32),
        ],
    )(flat, wr_pad, br_pad)


def _dense_body(x_ref, dm_ref, w1_ref, b1_ref, w2_ref, b2_ref, out_ref):
    e = pl.program_id(1)
    j = pl.program_id(2)
    dm = dm_ref[...]
    lane = jax.lax.broadcasted_iota(jnp.int32, dm.shape, 1)
    m = jnp.sum(jnp.where(lane == e, dm, 0.0), axis=1, keepdims=True)
    xm = x_ref[...] * m
    h = jnp.maximum(
        jnp.dot(xm, w1_ref[0], preferred_element_type=jnp.float32) + b1_ref[0],
        0.0)
    part = jnp.dot(h, w2_ref[0], preferred_element_type=jnp.float32)
    jfirst = (j == 0).astype(jnp.float32)
    total = m * part + jfirst * (m * b2_ref[0])
    first = jnp.logical_and(e == 0, j == 0)

    @pl.when(first)
    def _():
        out_ref[...] = total

    @pl.when(jnp.logical_not(first))
    def _():
        out_ref[...] = out_ref[...] + total


def _run_dense(flat, dm_pad, w1, b1, w2, b2):
    n, d = flat.shape
    ne, _, f = w1.shape
    bm = min(512, n)
    fc = min(1024, f)
    grid = (n // bm, ne, f // fc)
    return pl.pallas_call(
        _dense_body,
        grid=grid,
        in_specs=[
            pl.BlockSpec((bm, d), lambda i, e, j: (i, 0)),
            pl.BlockSpec((bm, _LANES), lambda i, e, j: (i, 0)),
            pl.BlockSpec((1, d, fc), lambda i, e, j: (e, 0, j)),
            pl.BlockSpec((1, 1, fc), lambda i, e, j: (e, 0, j)),
            pl.BlockSpec((1, fc, d), lambda i, e, j: (e, j, 0)),
            pl.BlockSpec((1, 1, d), lambda i, e, j: (e, 0, 0)),
        ],
        out_specs=pl.BlockSpec((bm, d), lambda i, e, j: (i, 0)),
        out_shape=jax.ShapeDtypeStruct((n, d), jnp.float32),
    )(flat, dm_pad, w1, b1.reshape(ne, 1, f), w2, b2.reshape(ne, 1, d))


def kernel(hidden_states, W_router, b_router, W1, b1, W2, b2):
    bsz, seq, d = hidden_states.shape
    n = bsz * seq
    ne = W_router.shape[1]
    flat = hidden_states.reshape(n, d)
    wr_pad = jnp.pad(W_router, ((0, 0), (0, _LANES - ne)))
    br_pad = jnp.pad(b_router, (0, _LANES - ne)).reshape(1, _LANES)

    (logits_pad, rw_pad, dm_pad, _eidx, _rank, _gate,
     _counts, aux) = _run_router(flat, wr_pad, br_pad)

    out = _run_dense(flat, dm_pad, W1, b1, W2, b2)

    return (out.reshape(bsz, seq, d),
            rw_pad[:, :ne].reshape(bsz, seq, ne),
            dm_pad[:, :ne].reshape(bsz, seq, ne),
            aux[0, 0],
            logits_pad[:, :ne].reshape(bsz, seq, ne))
